# baseline (speedup 1.0000x reference)
import functools

import jax
import jax.numpy as jnp
from jax import lax
from jax.experimental import pallas as pl
from jax.experimental.pallas import tpu as pltpu
from jax.experimental.pallas import tpu_sc as plsc

N_WORKERS = 32
CHUNK = 320
NBUF = 2
DP = 128


def _gather_call(n_per_w, d, idx, w):
    mesh = plsc.VectorSubcoreMesh(core_axis_name="c", subcore_axis_name="s")
    n_chunks = n_per_w // CHUNK

    @functools.partial(
        pl.kernel,
        mesh=mesh,
        out_type=jax.ShapeDtypeStruct((N_WORKERS * n_per_w, DP), jnp.float32),
        scratch_types=[
            pltpu.VMEM((n_per_w,), jnp.int32),
        ]
        + [pltpu.VMEM((CHUNK, DP), jnp.float32) for _ in range(NBUF)]
        + [pltpu.SemaphoreType.DMA for _ in range(2 * NBUF)],
        compiler_params=pltpu.CompilerParams(use_tc_tiling_on_sc=True),
    )
    def k(idx_hbm, table_hbm, out_hbm, idx_v, *bufs):
        rows = bufs[:NBUF]
        gsems = bufs[NBUF : 2 * NBUF]
        osems = bufs[2 * NBUF :]
        cid = lax.axis_index("c")
        sid = lax.axis_index("s")
        wid = sid * 2 + cid
        out_base = wid * n_per_w
        pltpu.sync_copy(idx_hbm.at[pl.ds(out_base, n_per_w)], idx_v)

        def out_slice(j):
            return out_hbm.at[pl.ds(out_base + j * CHUNK, CHUNK)]

        def idx_slice(j):
            return idx_v.at[pl.ds(j * CHUNK, CHUNK)]

        for b in range(NBUF):
            pltpu.async_copy(table_hbm.at[idx_slice(b)], rows[b], gsems[b])

        for j in range(n_chunks):
            b = j % NBUF
            pltpu.make_async_copy(
                table_hbm.at[idx_slice(j)], rows[b], gsems[b]
            ).wait()
            pltpu.async_copy(rows[b], out_slice(j), osems[b])
            nj = j + NBUF
            if nj < n_chunks:
                pltpu.make_async_copy(rows[b], out_slice(j), osems[b]).wait()
                pltpu.async_copy(table_hbm.at[idx_slice(nj)], rows[b], gsems[b])

        for j in range(n_chunks - NBUF, n_chunks):
            b = j % NBUF
            pltpu.make_async_copy(rows[b], out_slice(j), osems[b]).wait()

    return k(idx, w)


def kernel(inputs, w):
    b, h = inputs.shape
    v, d = w.shape
    n = b * h
    assert n % (N_WORKERS * CHUNK) == 0
    n_per_w = n // N_WORKERS
    idx = inputs.T.reshape(n).astype(jnp.int32)
    wp = jnp.pad(w, ((0, 0), (0, DP - d)))
    out = _gather_call(n_per_w, d, idx, wp)
    return out.reshape(h, b, DP)[:, :, :d].swapaxes(0, 1)


# speedup vs baseline: 1.0884x; 1.0884x over previous
import functools

import jax
import jax.numpy as jnp
from jax import lax
from jax.experimental import pallas as pl
from jax.experimental.pallas import tpu as pltpu
from jax.experimental.pallas import tpu_sc as plsc

N_WORKERS = 32
CHUNK = 640
NBUF = 2
DP = 128


def _gather_call(n_per_w, d, idx, w):
    mesh = plsc.VectorSubcoreMesh(core_axis_name="c", subcore_axis_name="s")
    n_chunks = n_per_w // CHUNK

    @functools.partial(
        pl.kernel,
        mesh=mesh,
        out_type=jax.ShapeDtypeStruct((N_WORKERS * n_per_w, DP), jnp.float32),
        scratch_types=[
            pltpu.VMEM((n_per_w,), jnp.int32),
        ]
        + [pltpu.VMEM((CHUNK, d), jnp.float32) for _ in range(NBUF)]
        + [pltpu.SemaphoreType.DMA for _ in range(2 * NBUF)],
        compiler_params=pltpu.CompilerParams(use_tc_tiling_on_sc=False),
    )
    def k(idx_hbm, table_hbm, out_hbm, idx_v, *bufs):
        rows = bufs[:NBUF]
        gsems = bufs[NBUF : 2 * NBUF]
        osems = bufs[2 * NBUF :]
        cid = lax.axis_index("c")
        sid = lax.axis_index("s")
        wid = sid * 2 + cid
        out_base = wid * n_per_w
        pltpu.sync_copy(idx_hbm.at[pl.ds(out_base, n_per_w)], idx_v)

        def out_slice(j):
            return out_hbm.at[pl.ds(out_base + j * CHUNK, CHUNK), pl.ds(0, d)]

        def idx_slice(j):
            return idx_v.at[pl.ds(j * CHUNK, CHUNK)]

        for b in range(NBUF):
            pltpu.async_copy(table_hbm.at[idx_slice(b)], rows[b], gsems[b])

        for j in range(n_chunks):
            b = j % NBUF
            pltpu.make_async_copy(
                table_hbm.at[idx_slice(j)], rows[b], gsems[b]
            ).wait()
            pltpu.async_copy(rows[b], out_slice(j), osems[b])
            nj = j + NBUF
            if nj < n_chunks:
                pltpu.make_async_copy(rows[b], out_slice(j), osems[b]).wait()
                pltpu.async_copy(table_hbm.at[idx_slice(nj)], rows[b], gsems[b])

        for j in range(n_chunks - NBUF, n_chunks):
            b = j % NBUF
            pltpu.make_async_copy(rows[b], out_slice(j), osems[b]).wait()

    return k(idx, w)


def kernel(inputs, w):
    b, h = inputs.shape
    v, d = w.shape
    n = b * h
    assert n % (N_WORKERS * CHUNK) == 0
    n_per_w = n // N_WORKERS
    idx = (inputs.T.reshape(n) * 2).astype(jnp.int32)
    wp = jnp.pad(w, ((0, 0), (0, DP - d))).reshape(2 * v, d)
    out = _gather_call(n_per_w, d, idx, wp)
    return out.reshape(h, b, DP)[:, :, :d].swapaxes(0, 1)
